# baseline (speedup 1.0000x reference)
import functools

import jax
import jax.numpy as jnp
from jax import lax
from jax.experimental import pallas as pl
from jax.experimental.pallas import tpu as pltpu
from jax.experimental.pallas import tpu_sc as plsc

NUM_ITEM = 1000000
EMBED_DIM = 32
BATCH = 16384

_PACK = 128 // EMBED_DIM
_NROW = NUM_ITEM // _PACK

_NC = 2
_NS = 16
_NW = _NC * _NS
_B_PER_W = BATCH // _NW
_CHUNK = 128
_NCHUNK = _B_PER_W // _CHUNK
_L = 16

_mesh = plsc.VectorSubcoreMesh(core_axis_name="c", subcore_axis_name="s")


@functools.partial(
    pl.kernel,
    mesh=_mesh,
    out_type=jax.ShapeDtypeStruct((BATCH, EMBED_DIM), jnp.float32),
    scratch_types=[
        pltpu.VMEM((_B_PER_W,), jnp.int32),
        pltpu.VMEM((_B_PER_W,), jnp.int32),
        pltpu.VMEM((_CHUNK, 128), jnp.float32),
        pltpu.VMEM((_CHUNK, 128), jnp.float32),
        pltpu.VMEM((_B_PER_W, EMBED_DIM), jnp.float32),
        pltpu.SemaphoreType.DMA,
        pltpu.SemaphoreType.DMA,
    ],
    compiler_params=pltpu.CompilerParams(
        use_tc_tiling_on_sc=True,
        disable_bounds_checks=True,
        disable_semaphore_checks=True,
    ),
)
def _gather_kernel(
    table_hbm, idx_hbm, out_hbm, idx_v, pidx_v, buf0_v, buf1_v, out_v, sem0, sem1
):
    wid = lax.axis_index("s") * _NC + lax.axis_index("c")
    base = wid * _B_PER_W
    pltpu.sync_copy(idx_hbm.at[pl.ds(base, _B_PER_W)], idx_v)

    for v in range(_B_PER_W // _L):
        pidx_v[pl.ds(v * _L, _L)] = lax.shift_right_logical(
            idx_v[pl.ds(v * _L, _L)], 2
        )

    bufs = [buf0_v, buf1_v]
    sems = [sem0, sem1]

    def fire(t):
        return pltpu.async_copy(
            table_hbm.at[pidx_v.at[pl.ds(t * _CHUNK, _CHUNK)]],
            bufs[t % 2],
            sems[t % 2],
        )

    def extract(t):
        buf = bufs[t % 2]

        def block_body(b, _):
            ivec = idx_v[pl.ds(t * _CHUNK + b * _L, _L)]
            for j in range(_L):
                r = b * _L + j
                o = (ivec[j] & (_PACK - 1)) * EMBED_DIM
                out_v[t * _CHUNK + r, pl.ds(0, _L)] = buf[r, pl.ds(o, _L)]
                out_v[t * _CHUNK + r, pl.ds(_L, _L)] = buf[
                    r, pl.ds(o + _L, _L)
                ]
            return 0

        lax.fori_loop(0, _CHUNK // _L, block_body, 0)

    pending = fire(0)
    for t in range(_NCHUNK):
        if t + 1 < _NCHUNK:
            nxt = fire(t + 1)
        pending.wait()
        extract(t)
        if t + 1 < _NCHUNK:
            pending = nxt

    pltpu.sync_copy(out_v, out_hbm.at[pl.ds(base, _B_PER_W)])


def kernel(item_fea, embedding_itemId):
    idx = item_fea[:, 0].astype(jnp.int32)
    table = embedding_itemId.reshape(_NROW, 128)
    return _gather_kernel(table, idx)


# speedup vs baseline: 1.6174x; 1.6174x over previous
import functools

import jax
import jax.numpy as jnp
from jax import lax
from jax.experimental import pallas as pl
from jax.experimental.pallas import tpu as pltpu
from jax.experimental.pallas import tpu_sc as plsc

NUM_ITEM = 1000000
EMBED_DIM = 32
BATCH = 16384

_NC = 2
_NS = 16
_NW = _NC * _NS
_B_PER_W = BATCH // _NW
_K = 64
_L = 16

_mesh = plsc.VectorSubcoreMesh(core_axis_name="c", subcore_axis_name="s")


@functools.partial(
    pl.kernel,
    mesh=_mesh,
    out_type=jax.ShapeDtypeStruct((BATCH, EMBED_DIM), jnp.float32),
    scratch_types=[
        pltpu.VMEM((_B_PER_W,), jnp.int32),
        pltpu.VMEM((_B_PER_W, EMBED_DIM), jnp.float32),
        pltpu.SemaphoreType.DMA,
    ],
    compiler_params=pltpu.CompilerParams(
        use_tc_tiling_on_sc=True,
        disable_bounds_checks=True,
        disable_semaphore_checks=True,
    ),
)
def _gather_kernel(table_hbm, idx_hbm, out_hbm, idx_v, rows_v, sem):
    wid = lax.axis_index("s") * _NC + lax.axis_index("c")
    base = wid * _B_PER_W
    pltpu.sync_copy(idx_hbm.at[pl.ds(base, _B_PER_W)], idx_v)

    for g in range(_B_PER_W // _K):
        copies = []
        for v in range(_K // _L):
            ivec = idx_v[pl.ds(g * _K + v * _L, _L)]
            for j in range(_L):
                r = g * _K + v * _L + j
                copies.append(
                    pltpu.async_copy(
                        table_hbm.at[pl.ds(ivec[j], 1)],
                        rows_v.at[pl.ds(r, 1)],
                        sem,
                    )
                )
        for c in copies:
            c.wait()

    pltpu.sync_copy(rows_v, out_hbm.at[pl.ds(base, _B_PER_W)])


def kernel(item_fea, embedding_itemId):
    idx = item_fea[:, 0].astype(jnp.int32)
    return _gather_kernel(embedding_itemId, idx)


# speedup vs baseline: 1.6251x; 1.0048x over previous
import functools

import jax
import jax.numpy as jnp
from jax import lax
from jax.experimental import pallas as pl
from jax.experimental.pallas import tpu as pltpu
from jax.experimental.pallas import tpu_sc as plsc

NUM_ITEM = 1000000
EMBED_DIM = 32
BATCH = 16384

_NC = 2
_NS = 16
_NW = _NC * _NS
_B_PER_W = BATCH // _NW
_K = 128
_L = 16

_mesh = plsc.VectorSubcoreMesh(core_axis_name="c", subcore_axis_name="s")


@functools.partial(
    pl.kernel,
    mesh=_mesh,
    out_type=jax.ShapeDtypeStruct((BATCH, EMBED_DIM), jnp.float32),
    scratch_types=[
        pltpu.VMEM((_B_PER_W,), jnp.int32),
        pltpu.VMEM((_B_PER_W, EMBED_DIM), jnp.float32),
        pltpu.SemaphoreType.DMA,
    ],
    compiler_params=pltpu.CompilerParams(
        use_tc_tiling_on_sc=True,
        disable_bounds_checks=True,
        disable_semaphore_checks=True,
    ),
)
def _gather_kernel(table_hbm, idx_hbm, out_hbm, idx_v, rows_v, sem):
    wid = lax.axis_index("s") * _NC + lax.axis_index("c")
    base = wid * _B_PER_W
    pltpu.sync_copy(idx_hbm.at[pl.ds(base, _B_PER_W)], idx_v)

    for g in range(_B_PER_W // _K):
        copies = []
        for v in range(_K // _L):
            ivec = idx_v[pl.ds(g * _K + v * _L, _L)]
            for j in range(_L):
                r = g * _K + v * _L + j
                copies.append(
                    pltpu.async_copy(
                        table_hbm.at[pl.ds(ivec[j], 1)],
                        rows_v.at[pl.ds(r, 1)],
                        sem,
                    )
                )
        for c in copies:
            c.wait()

    pltpu.sync_copy(rows_v, out_hbm.at[pl.ds(base, _B_PER_W)])


def kernel(item_fea, embedding_itemId):
    idx = item_fea[:, 0].astype(jnp.int32)
    return _gather_kernel(embedding_itemId, idx)
